# baseline (speedup 1.0000x reference)
import dataclasses
import functools

import jax
import jax.numpy as jnp
from jax import lax
from jax.experimental import pallas as pl
from jax.experimental.pallas import tpu as pltpu
from jax.experimental.pallas import tpu_sc as plsc

N_NODES = 10000
N_EDGES = 160000
D = 256
DH = 128

N_PAD = 10240
E_PAD = 163840
DUMMY = 10000

NC, NS = 2, 16
EB = 64

_MESH = plsc.VectorSubcoreMesh(
    core_axis_name="c", subcore_axis_name="s", num_cores=NC, num_subcores=NS
)

_SC_PARAMS = pltpu.CompilerParams()
if "needs_layout_passes" in pltpu.CompilerParams.__dataclass_fields__:
    _SC_PARAMS = dataclasses.replace(_SC_PARAMS, needs_layout_passes=False)
if "use_tc_tiling_on_sc" in pltpu.CompilerParams.__dataclass_fields__:
    _SC_PARAMS = dataclasses.replace(_SC_PARAMS, use_tc_tiling_on_sc=False)


def _proj_body(feat_ref, w_ref, b_ref, hn_ref, hv_ref):
    x = feat_ref[...]
    h = lax.dot_general(
        x, w_ref[...], (((1,), (1,)), ((), ())),
        precision=lax.Precision.HIGHEST,
        preferred_element_type=jnp.float32,
    ) + b_ref[...]
    norm = jnp.sqrt(jnp.sum(h * h, axis=1, keepdims=True))
    hn = h / jnp.maximum(norm, 1e-12)
    hn_ref[...] = hn.astype(jnp.bfloat16).reshape(256, 2, DH)
    hv_ref[0] = h[:, :DH].astype(jnp.bfloat16)
    hv_ref[1] = h[:, DH:].astype(jnp.bfloat16)


def _project(feat_p, W, b2):
    nblk = N_PAD // 256
    return pl.pallas_call(
        _proj_body,
        grid=(nblk,),
        in_specs=[
            pl.BlockSpec((256, D), lambda i: (i, 0)),
            pl.BlockSpec((D, D), lambda i: (0, 0)),
            pl.BlockSpec((1, D), lambda i: (0, 0)),
        ],
        out_specs=[
            pl.BlockSpec((256, 2, DH), lambda i: (i, 0, 0)),
            pl.BlockSpec((2, 256, DH), lambda i: (0, i, 0)),
        ],
        out_shape=[
            jax.ShapeDtypeStruct((N_PAD, 2, DH), jnp.bfloat16),
            jax.ShapeDtypeStruct((2, N_PAD, DH), jnp.bfloat16),
        ],
    )(feat_p, W, b2)


EBB = 32
EPT_B = E_PAD // (NC * NS)
NBLK_B = EPT_B // EBB
RPT_B = N_PAD // NS


def _edge_w_body(hn_hbm, src_hbm, dst_hbm, w_hbm, dpart_hbm,
                 sidx, didx, arows0, arows1, brows0, brows1, wall, dpart,
                 acc_ref, tbl_sh, sem0, sem1):
    cid = lax.axis_index("c")
    sid = lax.axis_index("s")
    wid = cid * NS + sid
    iota = lax.iota(jnp.int32, 16)
    sems = (sem0, sem1)
    abuf = (arows0, arows1)
    bbuf = (brows0, brows1)
    ebase = wid * EPT_B

    pltpu.sync_copy(src_hbm.at[pl.ds(ebase, EPT_B)], sidx)
    pltpu.sync_copy(dst_hbm.at[pl.ds(ebase, EPT_B)], didx)

    @pl.loop(0, N_PAD // 16)
    def _(i):
        dpart[pl.ds(i * 16, 16)] = jnp.zeros((16,), jnp.float32)

    rbase = sid * RPT_B

    def stage_rd(i, slot):
        pltpu.async_copy(hn_hbm.at[pl.ds(rbase + i * EBB, EBB)], abuf[slot],
                         sems[slot])

    def stage_wt(i, slot):
        pltpu.make_async_copy(hn_hbm.at[pl.ds(rbase + i * EBB, EBB)],
                              abuf[slot], sems[slot]).wait()
        pltpu.sync_copy(abuf[slot], tbl_sh.at[pl.ds(rbase + i * EBB, EBB)])

    nstage = RPT_B // EBB
    stage_rd(0, 0)

    @pl.loop(0, nstage // 2)
    def _(it):
        i = it * 2
        stage_rd(i + 1, 1)
        stage_wt(i, 0)

        @pl.when(it < nstage // 2 - 1)
        def _():
            stage_rd(i + 2, 0)

        stage_wt(i + 1, 1)

    plsc.subcore_barrier()

    def start_gathers(blk, slot):
        sl = pl.ds(blk * EBB, EBB)
        pltpu.async_copy(tbl_sh.at[sidx.at[sl]], abuf[slot], sems[slot])
        pltpu.async_copy(tbl_sh.at[didx.at[sl]], bbuf[slot], sems[slot])

    def wait_gathers(blk, slot):
        sl = pl.ds(blk * EBB, EBB)
        pltpu.make_async_copy(tbl_sh.at[sidx.at[sl]], abuf[slot],
                              sems[slot]).wait()
        pltpu.make_async_copy(tbl_sh.at[didx.at[sl]], bbuf[slot],
                              sems[slot]).wait()

    def compute(blk, slot):
        ar, br = abuf[slot], bbuf[slot]

        @pl.loop(0, EBB // 16)
        def _(g):
            @pl.loop(0, 16)
            def _(j):
                e = g * 16 + j
                acc = jnp.zeros((16,), jnp.float32)
                for h in range(2):
                    for c in range(DH // 32):
                        sl = pl.ds(c * 32, 32)
                        a0, a1 = plsc.unpack(
                            ar[e, h, sl], format=plsc.PackFormat.INTERLEAVED)
                        b0, b1 = plsc.unpack(
                            br[e, h, sl], format=plsc.PackFormat.INTERLEAVED)
                        acc = acc + a0 * b0 + a1 * b1
                s = jnp.sum(acc)
                acc_ref[...] = jnp.where(
                    iota == j, jnp.full((16,), s, jnp.float32), acc_ref[...])

            w16 = jnp.exp(acc_ref[...])
            off = blk * EBB + g * 16
            wall[pl.ds(off, 16)] = w16
            plsc.addupdate_scatter(dpart, [didx[pl.ds(off, 16)]], w16)

    start_gathers(0, 0)

    @pl.loop(0, NBLK_B // 2)
    def _(it):
        blk_a = it * 2
        blk_b = blk_a + 1
        start_gathers(blk_b, 1)
        wait_gathers(blk_a, 0)
        compute(blk_a, 0)

        @pl.when(it < NBLK_B // 2 - 1)
        def _():
            start_gathers(blk_a + 2, 0)

        wait_gathers(blk_b, 1)
        compute(blk_b, 1)

    pltpu.sync_copy(wall, w_hbm.at[pl.ds(ebase, EPT_B)])
    pltpu.sync_copy(dpart, dpart_hbm.at[wid])


def _edge_weights(hnorm, src, dst):
    kern = pl.kernel(
        _edge_w_body,
        out_type=(
            jax.ShapeDtypeStruct((E_PAD,), jnp.float32),
            jax.ShapeDtypeStruct((NC * NS, N_PAD), jnp.float32),
        ),
        mesh=_MESH,
        scratch_types=[
            pltpu.VMEM((EPT_B,), jnp.int32),
            pltpu.VMEM((EPT_B,), jnp.int32),
            pltpu.VMEM((EBB, 2, DH), jnp.bfloat16),
            pltpu.VMEM((EBB, 2, DH), jnp.bfloat16),
            pltpu.VMEM((EBB, 2, DH), jnp.bfloat16),
            pltpu.VMEM((EBB, 2, DH), jnp.bfloat16),
            pltpu.VMEM((EPT_B,), jnp.float32),
            pltpu.VMEM((N_PAD,), jnp.float32),
            pltpu.VMEM((16,), jnp.float32),
            pltpu.VMEM_SHARED((N_PAD, 2, DH), jnp.bfloat16),
            pltpu.SemaphoreType.DMA,
            pltpu.SemaphoreType.DMA,
        ],
        compiler_params=_SC_PARAMS,
    )
    return kern(hnorm, src, dst)


EPT_C = E_PAD // NS
EBC = 64
NBLK_C = EPT_C // EBC
RPT = N_PAD // NS


JU = 4


def _aggregate_body(tbl_hbm, src_hbm, dst2_hbm, w_hbm, out_hbm,
                    sidx, didx_s, rows0, rows1, gb0, gb1, wv, acc_sh,
                    sg0, sg1, ss0, ss1):
    cid = lax.axis_index("c")
    sid = lax.axis_index("s")
    rbuf = (rows0, rows1)
    gbuf = (gb0, gb1)
    gsem = (sg0, sg1)
    ssem = (ss0, ss1)

    ebase = sid * EPT_C
    pltpu.sync_copy(src_hbm.at[pl.ds(ebase, EPT_C)], sidx)
    off = cid * N_PAD

    @pl.loop(0, EPT_C // 16)
    def _(i):
        sl = pl.ds(i * 16, 16)
        sidx[sl] = sidx[sl] + off

    @pl.loop(0, EBC)
    def _(j):
        for k in range(DH // 16):
            rows0[j, pl.ds(k * 16, 16)] = jnp.zeros((16,), jnp.float32)

    @pl.loop(0, RPT // EBC)
    def _(i):
        pltpu.sync_copy(rows0, acc_sh.at[pl.ds(sid * RPT + i * EBC, EBC)])

    plsc.subcore_barrier()

    def start_gather(blk, slot):
        pltpu.async_copy(tbl_hbm.at[sidx.at[pl.ds(blk * EBC, EBC)]],
                         gbuf[slot], gsem[slot])
        pltpu.async_copy(dst2_hbm.at[sid * NBLK_C + blk], didx_s.at[slot],
                         gsem[slot])
        pltpu.async_copy(w_hbm.at[pl.ds(ebase + blk * EBC, EBC)],
                         wv.at[pl.ds(slot * EBC, EBC)], gsem[slot])

    def wait_gather(blk, slot):
        pltpu.make_async_copy(tbl_hbm.at[sidx.at[pl.ds(blk * EBC, EBC)]],
                              gbuf[slot], gsem[slot]).wait()
        pltpu.make_async_copy(dst2_hbm.at[sid * NBLK_C + blk],
                              didx_s.at[slot], gsem[slot]).wait()
        pltpu.make_async_copy(w_hbm.at[pl.ds(ebase + blk * EBC, EBC)],
                              wv.at[pl.ds(slot * EBC, EBC)],
                              gsem[slot]).wait()

    def scale(blk, slot):
        g = gbuf[slot]
        r = rbuf[slot]

        @pl.loop(0, EBC, step=JU)
        def _(j0):
            for dj in range(JU):
                j = j0 + dj
                cj = plsc.load_gather(
                    wv, [jnp.full((16,), slot * EBC + j, jnp.int32)])
                for k in range(DH // 32):
                    v0, v1 = plsc.unpack(g[j, pl.ds(k * 32, 32)],
                                         format=plsc.PackFormat.INTERLEAVED)
                    r[j, pl.ds(k * 16, 16)] = v0 * cj
                    r[j, pl.ds(64 + k * 16, 16)] = v1 * cj

    def start_scatter(blk, slot):
        pltpu.async_copy(rbuf[slot], acc_sh.at[didx_s.at[slot]], ssem[slot],
                         add=True)

    def drain_scatter(slot):
        pltpu.make_async_copy(rbuf[slot], acc_sh.at[didx_s.at[slot]],
                              ssem[slot]).wait()

    start_gather(0, 0)

    @pl.loop(0, NBLK_C // 2)
    def _(it):
        blk_a = it * 2
        blk_b = blk_a + 1
        wait_gather(blk_a, 0)
        scale(blk_a, 0)
        start_scatter(blk_a, 0)

        @pl.when(it > 0)
        def _():
            drain_scatter(1)

        start_gather(blk_b, 1)
        wait_gather(blk_b, 1)
        scale(blk_b, 1)
        start_scatter(blk_b, 1)
        drain_scatter(0)

        @pl.when(it < NBLK_C // 2 - 1)
        def _():
            start_gather(blk_a + 2, 0)

    drain_scatter(1)
    plsc.subcore_barrier()

    @pl.loop(0, RPT // EBC)
    def _(i):
        r0 = sid * RPT + i * EBC
        pltpu.sync_copy(acc_sh.at[pl.ds(r0, EBC)], rows0)
        pltpu.sync_copy(rows0, out_hbm.at[pl.ds(cid * N_PAD + r0, EBC)])


def _aggregate(table, src, dst2, w):
    kern = pl.kernel(
        _aggregate_body,
        out_type=jax.ShapeDtypeStruct((2 * N_PAD, DH), jnp.float32),
        mesh=_MESH,
        scratch_types=[
            pltpu.VMEM((EPT_C,), jnp.int32),
            pltpu.VMEM((2, EBC), jnp.int32),
            pltpu.VMEM((EBC, DH), jnp.float32),
            pltpu.VMEM((EBC, DH), jnp.float32),
            pltpu.VMEM((EBC, DH), jnp.bfloat16),
            pltpu.VMEM((EBC, DH), jnp.bfloat16),
            pltpu.VMEM((2 * EBC,), jnp.float32),
            pltpu.VMEM_SHARED((N_PAD, DH), jnp.float32),
            pltpu.SemaphoreType.DMA,
            pltpu.SemaphoreType.DMA,
            pltpu.SemaphoreType.DMA,
            pltpu.SemaphoreType.DMA,
        ],
        compiler_params=_SC_PARAMS,
    )
    return kern(table, src, dst2, w)


def _final_body(lo_ref, hi_ref, dp_ref, out_ref):
    den = jnp.maximum(jnp.sum(dp_ref[...], axis=0), 1e-38)
    num = jnp.concatenate([lo_ref[...], hi_ref[...]], axis=1)
    out_ref[...] = num / den[:, None]


def _finalize(num, dparts):
    nblk = N_PAD // 256
    return pl.pallas_call(
        _final_body,
        grid=(nblk,),
        in_specs=[
            pl.BlockSpec((256, DH), lambda i: (i, 0)),
            pl.BlockSpec((256, DH), lambda i: (nblk + i, 0)),
            pl.BlockSpec((NC * NS, 256), lambda i: (0, i)),
        ],
        out_specs=pl.BlockSpec((256, D), lambda i: (i, 0)),
        out_shape=jax.ShapeDtypeStruct((N_PAD, D), jnp.float32),
    )(num, num, dparts)


@jax.jit
def kernel(feat, edge_index, W, b):
    feat_p = jnp.pad(feat, ((0, N_PAD - N_NODES), (0, 0)))
    src = jnp.pad(edge_index[0], (0, E_PAD - N_EDGES), constant_values=DUMMY)
    dst = jnp.pad(edge_index[1], (0, E_PAD - N_EDGES), constant_values=DUMMY)
    b2 = b.reshape(1, D)

    hnorm, hv = _project(feat_p, W, b2)
    table = (hv.reshape(2, N_PAD, 2, DH // 2)
               .transpose(0, 1, 3, 2)
               .reshape(2 * N_PAD, DH))
    w, dparts = _edge_weights(hnorm, src, dst)
    num = _aggregate(table, src, dst.reshape(E_PAD // EBC, EBC), w)
    out = _finalize(num, dparts)
    return out[:N_NODES]


# speedup vs baseline: 1.1201x; 1.1201x over previous
import dataclasses
import functools

import jax
import jax.numpy as jnp
from jax import lax
from jax.experimental import pallas as pl
from jax.experimental.pallas import tpu as pltpu
from jax.experimental.pallas import tpu_sc as plsc

N_NODES = 10000
N_EDGES = 160000
D = 256
DH = 128

N_PAD = 10240
E_PAD = 163840
DUMMY = 10000

NC, NS = 2, 16
EB = 64

_MESH = plsc.VectorSubcoreMesh(
    core_axis_name="c", subcore_axis_name="s", num_cores=NC, num_subcores=NS
)

_SC_PARAMS = pltpu.CompilerParams()
if "needs_layout_passes" in pltpu.CompilerParams.__dataclass_fields__:
    _SC_PARAMS = dataclasses.replace(_SC_PARAMS, needs_layout_passes=False)
if "use_tc_tiling_on_sc" in pltpu.CompilerParams.__dataclass_fields__:
    _SC_PARAMS = dataclasses.replace(_SC_PARAMS, use_tc_tiling_on_sc=False)


def _proj_body(feat_ref, w_ref, b_ref, hn_ref, hv_ref):
    x = feat_ref[...]
    h = lax.dot_general(
        x, w_ref[...], (((1,), (1,)), ((), ())),
        precision=lax.Precision.HIGHEST,
        preferred_element_type=jnp.float32,
    ) + b_ref[...]
    norm = jnp.sqrt(jnp.sum(h * h, axis=1, keepdims=True))
    hn = h / jnp.maximum(norm, 1e-12)
    hn_ref[...] = hn.astype(jnp.bfloat16).reshape(256, 2, DH)
    hv_ref[0] = h[:, :DH]
    hv_ref[1] = h[:, DH:]


def _project(feat_p, W, b2):
    nblk = N_PAD // 256
    return pl.pallas_call(
        _proj_body,
        grid=(nblk,),
        in_specs=[
            pl.BlockSpec((256, D), lambda i: (i, 0)),
            pl.BlockSpec((D, D), lambda i: (0, 0)),
            pl.BlockSpec((1, D), lambda i: (0, 0)),
        ],
        out_specs=[
            pl.BlockSpec((256, 2, DH), lambda i: (i, 0, 0)),
            pl.BlockSpec((2, 256, DH), lambda i: (0, i, 0)),
        ],
        out_shape=[
            jax.ShapeDtypeStruct((N_PAD, 2, DH), jnp.bfloat16),
            jax.ShapeDtypeStruct((2, N_PAD, DH), jnp.float32),
        ],
    )(feat_p, W, b2)


EBB = 32
EPT_B = E_PAD // (NC * NS)
NBLK_B = EPT_B // EBB
RPT_B = N_PAD // NS


def _edge_w_body(hn_hbm, src_hbm, dst_hbm, w_hbm, dpart_hbm,
                 sidx, didx, arows0, arows1, brows0, brows1, wall, dpart,
                 acc_ref, tbl_sh, sem0, sem1):
    cid = lax.axis_index("c")
    sid = lax.axis_index("s")
    wid = cid * NS + sid
    iota = lax.iota(jnp.int32, 16)
    sems = (sem0, sem1)
    abuf = (arows0, arows1)
    bbuf = (brows0, brows1)
    ebase = wid * EPT_B

    pltpu.sync_copy(src_hbm.at[pl.ds(ebase, EPT_B)], sidx)
    pltpu.sync_copy(dst_hbm.at[pl.ds(ebase, EPT_B)], didx)

    @pl.loop(0, N_PAD // 16)
    def _(i):
        dpart[pl.ds(i * 16, 16)] = jnp.zeros((16,), jnp.float32)

    rbase = sid * RPT_B

    def stage_rd(i, slot):
        pltpu.async_copy(hn_hbm.at[pl.ds(rbase + i * EBB, EBB)], abuf[slot],
                         sems[slot])

    def stage_wt(i, slot):
        pltpu.make_async_copy(hn_hbm.at[pl.ds(rbase + i * EBB, EBB)],
                              abuf[slot], sems[slot]).wait()
        pltpu.sync_copy(abuf[slot], tbl_sh.at[pl.ds(rbase + i * EBB, EBB)])

    nstage = RPT_B // EBB
    stage_rd(0, 0)

    @pl.loop(0, nstage // 2)
    def _(it):
        i = it * 2
        stage_rd(i + 1, 1)
        stage_wt(i, 0)

        @pl.when(it < nstage // 2 - 1)
        def _():
            stage_rd(i + 2, 0)

        stage_wt(i + 1, 1)

    plsc.subcore_barrier()

    def start_gathers(blk, slot):
        sl = pl.ds(blk * EBB, EBB)
        pltpu.async_copy(tbl_sh.at[sidx.at[sl]], abuf[slot], sems[slot])
        pltpu.async_copy(tbl_sh.at[didx.at[sl]], bbuf[slot], sems[slot])

    def wait_gathers(blk, slot):
        sl = pl.ds(blk * EBB, EBB)
        pltpu.make_async_copy(tbl_sh.at[sidx.at[sl]], abuf[slot],
                              sems[slot]).wait()
        pltpu.make_async_copy(tbl_sh.at[didx.at[sl]], bbuf[slot],
                              sems[slot]).wait()

    def compute(blk, slot):
        ar, br = abuf[slot], bbuf[slot]

        @pl.loop(0, EBB // 16)
        def _(g):
            @pl.loop(0, 16)
            def _(j):
                e = g * 16 + j
                acc = jnp.zeros((16,), jnp.float32)
                for h in range(2):
                    for c in range(DH // 32):
                        sl = pl.ds(c * 32, 32)
                        a0, a1 = plsc.unpack(
                            ar[e, h, sl], format=plsc.PackFormat.INTERLEAVED)
                        b0, b1 = plsc.unpack(
                            br[e, h, sl], format=plsc.PackFormat.INTERLEAVED)
                        acc = acc + a0 * b0 + a1 * b1
                s = jnp.sum(acc)
                acc_ref[...] = jnp.where(
                    iota == j, jnp.full((16,), s, jnp.float32), acc_ref[...])

            w16 = jnp.exp(acc_ref[...])
            off = blk * EBB + g * 16
            wall[pl.ds(off, 16)] = w16
            plsc.addupdate_scatter(dpart, [didx[pl.ds(off, 16)]], w16)

    start_gathers(0, 0)

    @pl.loop(0, NBLK_B // 2)
    def _(it):
        blk_a = it * 2
        blk_b = blk_a + 1
        start_gathers(blk_b, 1)
        wait_gathers(blk_a, 0)
        compute(blk_a, 0)

        @pl.when(it < NBLK_B // 2 - 1)
        def _():
            start_gathers(blk_a + 2, 0)

        wait_gathers(blk_b, 1)
        compute(blk_b, 1)

    pltpu.sync_copy(wall, w_hbm.at[pl.ds(ebase, EPT_B)])
    pltpu.sync_copy(dpart, dpart_hbm.at[wid])


def _edge_weights(hnorm, src, dst):
    kern = pl.kernel(
        _edge_w_body,
        out_type=(
            jax.ShapeDtypeStruct((E_PAD,), jnp.float32),
            jax.ShapeDtypeStruct((NC * NS, N_PAD), jnp.float32),
        ),
        mesh=_MESH,
        scratch_types=[
            pltpu.VMEM((EPT_B,), jnp.int32),
            pltpu.VMEM((EPT_B,), jnp.int32),
            pltpu.VMEM((EBB, 2, DH), jnp.bfloat16),
            pltpu.VMEM((EBB, 2, DH), jnp.bfloat16),
            pltpu.VMEM((EBB, 2, DH), jnp.bfloat16),
            pltpu.VMEM((EBB, 2, DH), jnp.bfloat16),
            pltpu.VMEM((EPT_B,), jnp.float32),
            pltpu.VMEM((N_PAD,), jnp.float32),
            pltpu.VMEM((16,), jnp.float32),
            pltpu.VMEM_SHARED((N_PAD, 2, DH), jnp.bfloat16),
            pltpu.SemaphoreType.DMA,
            pltpu.SemaphoreType.DMA,
        ],
        compiler_params=_SC_PARAMS,
    )
    return kern(hnorm, src, dst)


EPT_C = E_PAD // NS
EBC = 128
NBLK_C = EPT_C // EBC
RPT = N_PAD // NS


JU = 4


def _aggregate_body(tbl_hbm, src_hbm, dst2_hbm, w_hbm, out_hbm,
                    sidx, didx_s, rows0, rows1, wv, acc_sh,
                    sg0, sg1, ss0, ss1):
    cid = lax.axis_index("c")
    sid = lax.axis_index("s")
    rbuf = (rows0, rows1)
    gsem = (sg0, sg1)
    ssem = (ss0, ss1)

    ebase = sid * EPT_C
    pltpu.sync_copy(src_hbm.at[pl.ds(ebase, EPT_C)], sidx)
    off = cid * N_PAD

    @pl.loop(0, EPT_C // 16)
    def _(i):
        sl = pl.ds(i * 16, 16)
        sidx[sl] = sidx[sl] + off

    @pl.loop(0, EBC)
    def _(j):
        for k in range(DH // 16):
            rows0[j, pl.ds(k * 16, 16)] = jnp.zeros((16,), jnp.float32)

    @pl.loop(0, RPT // EBC)
    def _(i):
        pltpu.sync_copy(rows0, acc_sh.at[pl.ds(sid * RPT + i * EBC, EBC)])

    plsc.subcore_barrier()

    def start_gather(blk, slot):
        pltpu.async_copy(tbl_hbm.at[sidx.at[pl.ds(blk * EBC, EBC)]],
                         rbuf[slot], gsem[slot])
        pltpu.async_copy(dst2_hbm.at[sid * NBLK_C + blk], didx_s.at[slot],
                         gsem[slot])
        pltpu.async_copy(w_hbm.at[pl.ds(ebase + blk * EBC, EBC)],
                         wv.at[pl.ds(slot * EBC, EBC)], gsem[slot])

    def wait_gather(blk, slot):
        pltpu.make_async_copy(tbl_hbm.at[sidx.at[pl.ds(blk * EBC, EBC)]],
                              rbuf[slot], gsem[slot]).wait()
        pltpu.make_async_copy(dst2_hbm.at[sid * NBLK_C + blk],
                              didx_s.at[slot], gsem[slot]).wait()
        pltpu.make_async_copy(w_hbm.at[pl.ds(ebase + blk * EBC, EBC)],
                              wv.at[pl.ds(slot * EBC, EBC)],
                              gsem[slot]).wait()

    def scale(blk, slot):
        r = rbuf[slot]

        @pl.loop(0, EBC, step=JU)
        def _(j0):
            for dj in range(JU):
                j = j0 + dj
                cj = plsc.load_gather(
                    wv, [jnp.full((16,), slot * EBC + j, jnp.int32)])
                for k in range(DH // 16):
                    sl = pl.ds(k * 16, 16)
                    r[j, sl] = r[j, sl] * cj

    def start_scatter(blk, slot):
        pltpu.async_copy(rbuf[slot], acc_sh.at[didx_s.at[slot]], ssem[slot],
                         add=True)

    def drain_scatter(slot):
        pltpu.make_async_copy(rbuf[slot], acc_sh.at[didx_s.at[slot]],
                              ssem[slot]).wait()

    start_gather(0, 0)

    @pl.loop(0, NBLK_C // 2)
    def _(it):
        blk_a = it * 2
        blk_b = blk_a + 1
        wait_gather(blk_a, 0)
        scale(blk_a, 0)
        start_scatter(blk_a, 0)

        @pl.when(it > 0)
        def _():
            drain_scatter(1)

        start_gather(blk_b, 1)
        wait_gather(blk_b, 1)
        scale(blk_b, 1)
        start_scatter(blk_b, 1)
        drain_scatter(0)

        @pl.when(it < NBLK_C // 2 - 1)
        def _():
            start_gather(blk_a + 2, 0)

    drain_scatter(1)
    plsc.subcore_barrier()

    @pl.loop(0, RPT // EBC)
    def _(i):
        r0 = sid * RPT + i * EBC
        pltpu.sync_copy(acc_sh.at[pl.ds(r0, EBC)], rows0)
        pltpu.sync_copy(rows0, out_hbm.at[pl.ds(cid * N_PAD + r0, EBC)])


def _aggregate(table, src, dst2, w):
    kern = pl.kernel(
        _aggregate_body,
        out_type=jax.ShapeDtypeStruct((2 * N_PAD, DH), jnp.float32),
        mesh=_MESH,
        scratch_types=[
            pltpu.VMEM((EPT_C,), jnp.int32),
            pltpu.VMEM((2, EBC), jnp.int32),
            pltpu.VMEM((EBC, DH), jnp.float32),
            pltpu.VMEM((EBC, DH), jnp.float32),
            pltpu.VMEM((2 * EBC,), jnp.float32),
            pltpu.VMEM_SHARED((N_PAD, DH), jnp.float32),
            pltpu.SemaphoreType.DMA,
            pltpu.SemaphoreType.DMA,
            pltpu.SemaphoreType.DMA,
            pltpu.SemaphoreType.DMA,
        ],
        compiler_params=_SC_PARAMS,
    )
    return kern(table, src, dst2, w)


def _final_body(lo_ref, hi_ref, dp_ref, out_ref):
    den = jnp.maximum(jnp.sum(dp_ref[...], axis=0), 1e-38)
    num = jnp.concatenate([lo_ref[...], hi_ref[...]], axis=1)
    out_ref[...] = num / den[:, None]


def _finalize(num, dparts):
    nblk = N_PAD // 256
    return pl.pallas_call(
        _final_body,
        grid=(nblk,),
        in_specs=[
            pl.BlockSpec((256, DH), lambda i: (i, 0)),
            pl.BlockSpec((256, DH), lambda i: (nblk + i, 0)),
            pl.BlockSpec((NC * NS, 256), lambda i: (0, i)),
        ],
        out_specs=pl.BlockSpec((256, D), lambda i: (i, 0)),
        out_shape=jax.ShapeDtypeStruct((N_PAD, D), jnp.float32),
    )(num, num, dparts)


@jax.jit
def kernel(feat, edge_index, W, b):
    feat_p = jnp.pad(feat, ((0, N_PAD - N_NODES), (0, 0)))
    src = jnp.pad(edge_index[0], (0, E_PAD - N_EDGES), constant_values=DUMMY)
    dst = jnp.pad(edge_index[1], (0, E_PAD - N_EDGES), constant_values=DUMMY)
    b2 = b.reshape(1, D)

    hnorm, hv = _project(feat_p, W, b2)
    table = hv.reshape(2 * N_PAD, DH)
    w, dparts = _edge_weights(hnorm, src, dst)
    num = _aggregate(table, src, dst.reshape(E_PAD // EBC, EBC), w)
    out = _finalize(num, dparts)
    return out[:N_NODES]
